# baseline (device time: 185290 ns/iter reference)
import jax
import jax.numpy as jnp
from jax import lax
from jax.experimental import pallas as pl
from jax.experimental.pallas import tpu as pltpu

N_DEV = 8


def kernel(table, idx):
    v_per, d = table.shape
    n = idx.shape[0]
    n_chunks = 8
    c_rows = n // n_chunks

    idx_mat = idx.reshape(n_chunks, c_rows).T

    def body(idx_ref, table_ref, out_ref, comm_ref, send_sems, recv_sems):
        my = lax.axis_index("i")
        left = lax.rem(my + N_DEV - 1, N_DEV)
        right = lax.rem(my + 1, N_DEV)

        barrier = pltpu.get_barrier_semaphore()
        for nbr in (left, right):
            pl.semaphore_signal(
                barrier, inc=1,
                device_id=(nbr,), device_id_type=pl.DeviceIdType.MESH,
            )
        pl.semaphore_wait(barrier, 2)

        base = my * v_per
        tbl = table_ref[:, :]
        iota = lax.broadcasted_iota(jnp.int32, (c_rows, v_per), 1)
        for c in range(n_chunks):
            local = idx_ref[:, c:c + 1] - base
            oh = (local == iota).astype(jnp.float32)
            part = jnp.dot(oh, tbl, preferred_element_type=jnp.float32)
            comm_ref[0, c * c_rows:(c + 1) * c_rows, :] = part
        out_ref[:, :] = comm_ref[0]

        for h in range(N_DEV - 1):
            rdma = pltpu.make_async_remote_copy(
                src_ref=comm_ref.at[h],
                dst_ref=comm_ref.at[h + 1],
                send_sem=send_sems.at[h],
                recv_sem=recv_sems.at[h + 1],
                device_id=(right,),
                device_id_type=pl.DeviceIdType.MESH,
            )
            rdma.start()
            rdma.wait()
            out_ref[:, :] += comm_ref[h + 1]

    return pl.pallas_call(
        body,
        out_shape=jax.ShapeDtypeStruct((n, d), jnp.float32),
        in_specs=[
            pl.BlockSpec(memory_space=pltpu.VMEM),
            pl.BlockSpec(memory_space=pltpu.VMEM),
        ],
        out_specs=pl.BlockSpec(memory_space=pltpu.VMEM),
        scratch_shapes=[
            pltpu.VMEM((N_DEV, n, d), jnp.float32),
            pltpu.SemaphoreType.DMA((N_DEV,)),
            pltpu.SemaphoreType.DMA((N_DEV,)),
        ],
        compiler_params=pltpu.CompilerParams(collective_id=0),
    )(idx_mat, table)


# device time: 31601 ns/iter; 5.8634x vs baseline; 5.8634x over previous
import jax
import jax.numpy as jnp
from jax import lax
from jax.experimental import pallas as pl
from jax.experimental.pallas import tpu as pltpu

N_DEV = 8


def kernel(table, idx):
    v_per, d = table.shape
    n = idx.shape[0]
    c_rows = n // N_DEV

    idx_mat = idx.reshape(N_DEV, c_rows).T

    def body(idx_ref, table_ref, out_ref,
             p_ref, rs_ref, send1, rs_sems, send2, ag_sems):
        my = lax.axis_index("i")

        barrier = pltpu.get_barrier_semaphore()
        for off in range(1, N_DEV):
            peer = lax.rem(my + off, N_DEV)
            pl.semaphore_signal(
                barrier, inc=1,
                device_id=(peer,), device_id_type=pl.DeviceIdType.MESH,
            )
        pl.semaphore_wait(barrier, N_DEV - 1)

        base = my * v_per
        tbl = table_ref[:, :].astype(jnp.bfloat16)
        iota = lax.broadcasted_iota(jnp.int32, (c_rows, v_per), 1)
        for c in range(N_DEV):
            local = idx_ref[:, c:c + 1] - base
            oh = (local == iota).astype(jnp.bfloat16)
            part = jnp.dot(oh, tbl, preferred_element_type=jnp.float32)
            p_ref[c] = part.astype(jnp.bfloat16)

        sends1 = []
        for off in range(1, N_DEV):
            peer = lax.rem(my + off, N_DEV)
            rdma = pltpu.make_async_remote_copy(
                src_ref=p_ref.at[peer],
                dst_ref=rs_ref.at[my],
                send_sem=send1.at[off],
                recv_sem=rs_sems.at[my],
                device_id=(peer,),
                device_id_type=pl.DeviceIdType.MESH,
            )
            rdma.start()
            sends1.append(rdma)
        rs_ref[my] = p_ref[my]
        for off in range(1, N_DEV):
            peer = lax.rem(my + off, N_DEV)
            recv = pltpu.make_async_remote_copy(
                src_ref=p_ref.at[0],
                dst_ref=rs_ref.at[peer],
                send_sem=send1.at[0],
                recv_sem=rs_sems.at[peer],
                device_id=(peer,),
                device_id_type=pl.DeviceIdType.MESH,
            )
            recv.wait_recv()
        for rdma in sends1:
            rdma.wait_send()

        acc = rs_ref[0].astype(jnp.float32)
        for s in range(1, N_DEV):
            acc = acc + rs_ref[s].astype(jnp.float32)
        p_ref[my] = acc.astype(jnp.bfloat16)

        sends2 = []
        for off in range(1, N_DEV):
            peer = lax.rem(my + off, N_DEV)
            rdma = pltpu.make_async_remote_copy(
                src_ref=p_ref.at[my],
                dst_ref=p_ref.at[my],
                send_sem=send2.at[off],
                recv_sem=ag_sems.at[my],
                device_id=(peer,),
                device_id_type=pl.DeviceIdType.MESH,
            )
            rdma.start()
            sends2.append(rdma)
        for off in range(1, N_DEV):
            peer = lax.rem(my + off, N_DEV)
            recv = pltpu.make_async_remote_copy(
                src_ref=p_ref.at[0],
                dst_ref=p_ref.at[peer],
                send_sem=send2.at[0],
                recv_sem=ag_sems.at[peer],
                device_id=(peer,),
                device_id_type=pl.DeviceIdType.MESH,
            )
            recv.wait_recv()
        for rdma in sends2:
            rdma.wait_send()

        out_ref[:, :] = p_ref[:].reshape(n, d).astype(jnp.float32)

    return pl.pallas_call(
        body,
        out_shape=jax.ShapeDtypeStruct((n, d), jnp.float32),
        in_specs=[
            pl.BlockSpec(memory_space=pltpu.VMEM),
            pl.BlockSpec(memory_space=pltpu.VMEM),
        ],
        out_specs=pl.BlockSpec(memory_space=pltpu.VMEM),
        scratch_shapes=[
            pltpu.VMEM((N_DEV, c_rows, d), jnp.bfloat16),
            pltpu.VMEM((N_DEV, c_rows, d), jnp.bfloat16),
            pltpu.SemaphoreType.DMA((N_DEV,)),
            pltpu.SemaphoreType.DMA((N_DEV,)),
            pltpu.SemaphoreType.DMA((N_DEV,)),
            pltpu.SemaphoreType.DMA((N_DEV,)),
        ],
        compiler_params=pltpu.CompilerParams(collective_id=0),
    )(idx_mat, table)


# device time: 27648 ns/iter; 6.7018x vs baseline; 1.1430x over previous
import jax
import jax.numpy as jnp
from jax import lax
from jax.experimental import pallas as pl
from jax.experimental.pallas import tpu as pltpu

N_DEV = 8


def kernel(table, idx):
    v_per, d = table.shape
    n = idx.shape[0]
    c_rows = n // N_DEV

    my_out = lax.axis_index("i")
    idx_mat = idx.reshape(N_DEV, c_rows).T
    order = (my_out + 1 + jnp.arange(N_DEV, dtype=jnp.int32)) % N_DEV
    idx_rot = jnp.take(idx_mat, order, axis=1)

    def body(idx_ref, table_ref, out_ref,
             p_ref, rs_ref, send1, rs_sems, send2, ag_sems):
        my = lax.axis_index("i")

        barrier = pltpu.get_barrier_semaphore()
        for off in range(1, N_DEV):
            peer = lax.rem(my + off, N_DEV)
            pl.semaphore_signal(
                barrier, inc=1,
                device_id=(peer,), device_id_type=pl.DeviceIdType.MESH,
            )
        pl.semaphore_wait(barrier, N_DEV - 1)

        base = my * v_per
        tbl = table_ref[:, :].astype(jnp.bfloat16)
        iota = lax.broadcasted_iota(jnp.int32, (c_rows, v_per), 1)
        sends1 = []
        for t in range(N_DEV):
            c = lax.rem(my + 1 + t, N_DEV)
            local = idx_ref[:, t:t + 1] - base
            oh = (local == iota).astype(jnp.bfloat16)
            part = jnp.dot(oh, tbl, preferred_element_type=jnp.float32)
            p_ref[c] = part.astype(jnp.bfloat16)
            if t < N_DEV - 1:
                rdma = pltpu.make_async_remote_copy(
                    src_ref=p_ref.at[c],
                    dst_ref=rs_ref.at[my],
                    send_sem=send1.at[t],
                    recv_sem=rs_sems.at[my],
                    device_id=(c,),
                    device_id_type=pl.DeviceIdType.MESH,
                )
                rdma.start()
                sends1.append(rdma)
        rs_ref[my] = p_ref[my]

        for s in range(N_DEV):
            @pl.when(s != my)
            def _():
                recv = pltpu.make_async_remote_copy(
                    src_ref=p_ref.at[0],
                    dst_ref=rs_ref.at[s],
                    send_sem=send1.at[0],
                    recv_sem=rs_sems.at[s],
                    device_id=(my,),
                    device_id_type=pl.DeviceIdType.MESH,
                )
                recv.wait_recv()
        for rdma in sends1:
            rdma.wait_send()

        acc = rs_ref[0].astype(jnp.float32)
        for s in range(1, N_DEV):
            acc = acc + rs_ref[s].astype(jnp.float32)
        out_ref[pl.ds(my * c_rows, c_rows), :] = acc
        p_ref[my] = acc.astype(jnp.bfloat16)

        sends2 = []
        for t in range(N_DEV - 1):
            peer = lax.rem(my + 1 + t, N_DEV)
            rdma = pltpu.make_async_remote_copy(
                src_ref=p_ref.at[my],
                dst_ref=p_ref.at[my],
                send_sem=send2.at[t],
                recv_sem=ag_sems.at[my],
                device_id=(peer,),
                device_id_type=pl.DeviceIdType.MESH,
            )
            rdma.start()
            sends2.append(rdma)
        for s in range(N_DEV):
            @pl.when(s != my)
            def _():
                recv = pltpu.make_async_remote_copy(
                    src_ref=p_ref.at[0],
                    dst_ref=p_ref.at[s],
                    send_sem=send2.at[0],
                    recv_sem=ag_sems.at[s],
                    device_id=(my,),
                    device_id_type=pl.DeviceIdType.MESH,
                )
                recv.wait_recv()
                out_ref[s * c_rows:(s + 1) * c_rows, :] = (
                    p_ref[s].astype(jnp.float32)
                )
        for rdma in sends2:
            rdma.wait_send()

    return pl.pallas_call(
        body,
        out_shape=jax.ShapeDtypeStruct((n, d), jnp.float32),
        in_specs=[
            pl.BlockSpec(memory_space=pltpu.VMEM),
            pl.BlockSpec(memory_space=pltpu.VMEM),
        ],
        out_specs=pl.BlockSpec(memory_space=pltpu.VMEM),
        scratch_shapes=[
            pltpu.VMEM((N_DEV, c_rows, d), jnp.bfloat16),
            pltpu.VMEM((N_DEV, c_rows, d), jnp.bfloat16),
            pltpu.SemaphoreType.DMA((N_DEV,)),
            pltpu.SemaphoreType.DMA((N_DEV,)),
            pltpu.SemaphoreType.DMA((N_DEV,)),
            pltpu.SemaphoreType.DMA((N_DEV,)),
        ],
        compiler_params=pltpu.CompilerParams(collective_id=0),
    )(idx_rot, table)
